# baseline (speedup 1.0000x reference)
import functools

import jax
import jax.numpy as jnp
from jax import lax
from jax.experimental import pallas as pl
from jax.experimental.pallas import tpu as pltpu
from jax.experimental.pallas import tpu_sc as plsc

VOCAB = 100000
EMBED_DIM = 64
BATCH = 1024


_NC, _NS = 2, 16
_NW = _NC * _NS
_B_PER_W = BATCH // _NW


def _sc_gather_body(table_hbm, idx_hbm, out_hbm, idx_v, rows_v, sem):
    wid = lax.axis_index("s") * _NC + lax.axis_index("c")
    base = wid * _B_PER_W
    pltpu.sync_copy(idx_hbm.at[pl.ds(base, _B_PER_W)], idx_v)
    pltpu.async_copy(table_hbm.at[idx_v], rows_v, sem).wait()
    pltpu.sync_copy(rows_v, out_hbm.at[pl.ds(base, _B_PER_W)])


def _sc_gather(table, idx):
    mesh = plsc.VectorSubcoreMesh(core_axis_name="c", subcore_axis_name="s")
    return pl.kernel(
        _sc_gather_body,
        out_type=jax.ShapeDtypeStruct((BATCH, EMBED_DIM), jnp.float32),
        mesh=mesh,
        scratch_types=[
            pltpu.VMEM((_B_PER_W,), jnp.int32),
            pltpu.VMEM((_B_PER_W, EMBED_DIM), jnp.float32),
            pltpu.SemaphoreType.DMA,
        ],
        compiler_params=pltpu.CompilerParams(use_tc_tiling_on_sc=False),
    )(table, idx)


_N_BLK = 2048
_NB = pl.cdiv(VOCAB, _N_BLK)
_LAST = VOCAB - (_NB - 1) * _N_BLK
_KW = 2
_KO = 6
_BPAD = _NB * _N_BLK


def _proj_body(x_ref, b_ref, w_hbm, o_hbm, w_bufs, o_bufs, w_sem, o_sem):
    def rows(j):
        return _N_BLK if j < _NB - 1 else _LAST

    def start_w(j):
        pltpu.make_async_copy(
            w_hbm.at[pl.ds(j * _N_BLK, rows(j))],
            w_bufs.at[j % _KW, pl.ds(0, rows(j))],
            w_sem.at[j % _KW],
        ).start()

    def wait_w(j):
        pltpu.make_async_copy(
            w_hbm.at[pl.ds(j * _N_BLK, rows(j))],
            w_bufs.at[j % _KW, pl.ds(0, rows(j))],
            w_sem.at[j % _KW],
        ).wait()

    def out_copy(j):
        return pltpu.make_async_copy(
            o_bufs.at[j % _KO, pl.ds(0, rows(j))],
            o_hbm.at[pl.ds(j * _N_BLK, rows(j))],
            o_sem.at[j % _KO],
        )

    for j in range(min(_KW, _NB)):
        start_w(j)

    x = x_ref[...]
    vrow = lax.broadcasted_iota(jnp.int32, (_N_BLK, 16), 0)
    sel = (vrow // 128 == lax.broadcasted_iota(jnp.int32, (_N_BLK, 16), 1))
    s_mat = sel.astype(jnp.float32)
    vlane = lax.broadcasted_iota(jnp.int32, (_N_BLK, 128), 0) % 128
    lane_mask = (
        vlane == lax.broadcasted_iota(jnp.int32, (_N_BLK, 128), 1)
    ).astype(jnp.float32)

    for j in range(_NB):
        wait_w(j)
        if j >= _KO:
            out_copy(j - _KO).wait()
        chunk = b_ref[pl.ds(j * (_N_BLK // 128), _N_BLK // 128), :]
        spread = lax.dot_general(
            s_mat, chunk, (((1,), (0,)), ((), ())),
            preferred_element_type=jnp.float32,
        )
        bias_col = jnp.sum(spread * lane_mask, axis=1, keepdims=True)
        o_bufs[j % _KO] = (
            lax.dot_general(
                w_bufs[j % _KW], x, (((1,), (1,)), ((), ())),
                preferred_element_type=jnp.float32,
            )
            + bias_col
        )
        out_copy(j).start()
        if j + _KW < _NB:
            start_w(j + _KW)

    for j in range(max(_NB - _KO, 0), _NB):
        out_copy(j).wait()


def _tc_project(x, W, b_mat):
    return pl.pallas_call(
        _proj_body,
        in_specs=[
            pl.BlockSpec(memory_space=pltpu.VMEM),
            pl.BlockSpec(memory_space=pltpu.VMEM),
            pl.BlockSpec(memory_space=pltpu.HBM),
        ],
        out_specs=pl.BlockSpec(memory_space=pltpu.HBM),
        out_shape=jax.ShapeDtypeStruct((VOCAB, BATCH), jnp.float32),
        scratch_shapes=[
            pltpu.VMEM((_KW, _N_BLK, EMBED_DIM), jnp.float32),
            pltpu.VMEM((_KO, _N_BLK, BATCH), jnp.float32),
            pltpu.SemaphoreType.DMA((_KW,)),
            pltpu.SemaphoreType.DMA((_KO,)),
        ],
    )(x, b_mat, W)


@jax.jit
def kernel(prev_relation_id, relation_embeddings, W, b):
    idx = prev_relation_id.astype(jnp.int32)
    rows = _sc_gather(relation_embeddings, idx)
    b_mat = jnp.pad(b, (0, _BPAD - VOCAB)).reshape(_BPAD // 128, 128)
    scores_t = _tc_project(rows, W, b_mat)
    return scores_t.T


# speedup vs baseline: 1.0085x; 1.0085x over previous
import functools

import jax
import jax.numpy as jnp
from jax import lax
from jax.experimental import pallas as pl
from jax.experimental.pallas import tpu as pltpu
from jax.experimental.pallas import tpu_sc as plsc

VOCAB = 100000
EMBED_DIM = 64
BATCH = 1024


_NC, _NS = 2, 16
_NW = _NC * _NS
_B_PER_W = BATCH // _NW


def _sc_gather_body(table_hbm, idx_hbm, out_hbm, idx_v, rows_v, sem):
    wid = lax.axis_index("s") * _NC + lax.axis_index("c")
    base = wid * _B_PER_W
    pltpu.sync_copy(idx_hbm.at[pl.ds(base, _B_PER_W)], idx_v)
    pltpu.async_copy(table_hbm.at[idx_v], rows_v, sem).wait()
    pltpu.sync_copy(rows_v, out_hbm.at[pl.ds(base, _B_PER_W)])


def _sc_gather(table, idx):
    mesh = plsc.VectorSubcoreMesh(core_axis_name="c", subcore_axis_name="s")
    return pl.kernel(
        _sc_gather_body,
        out_type=jax.ShapeDtypeStruct((BATCH, EMBED_DIM), jnp.float32),
        mesh=mesh,
        scratch_types=[
            pltpu.VMEM((_B_PER_W,), jnp.int32),
            pltpu.VMEM((_B_PER_W, EMBED_DIM), jnp.float32),
            pltpu.SemaphoreType.DMA,
        ],
        compiler_params=pltpu.CompilerParams(use_tc_tiling_on_sc=False),
    )(table, idx)


_N_BLK = 4096
_NB = pl.cdiv(VOCAB, _N_BLK)
_LAST = VOCAB - (_NB - 1) * _N_BLK
_KW = 2
_KO = 2
_BPAD = _NB * _N_BLK


def _proj_body(x_ref, b_ref, w_hbm, o_hbm, w_bufs, o_bufs, w_sem, o_sem):
    def rows(j):
        return _N_BLK if j < _NB - 1 else _LAST

    def start_w(j):
        pltpu.make_async_copy(
            w_hbm.at[pl.ds(j * _N_BLK, rows(j))],
            w_bufs.at[j % _KW, pl.ds(0, rows(j))],
            w_sem.at[j % _KW],
        ).start()

    def wait_w(j):
        pltpu.make_async_copy(
            w_hbm.at[pl.ds(j * _N_BLK, rows(j))],
            w_bufs.at[j % _KW, pl.ds(0, rows(j))],
            w_sem.at[j % _KW],
        ).wait()

    def out_copy(j):
        return pltpu.make_async_copy(
            o_bufs.at[j % _KO, pl.ds(0, rows(j))],
            o_hbm.at[pl.ds(j * _N_BLK, rows(j))],
            o_sem.at[j % _KO],
        )

    for j in range(min(_KW, _NB)):
        start_w(j)

    x = x_ref[...]
    nck = _N_BLK // 128
    vrow = lax.broadcasted_iota(jnp.int32, (_N_BLK, nck), 0)
    sel = (vrow // 128 == lax.broadcasted_iota(jnp.int32, (_N_BLK, nck), 1))
    s_mat = sel.astype(jnp.float32)
    vlane = lax.broadcasted_iota(jnp.int32, (_N_BLK, 128), 0) % 128
    lane_mask = (
        vlane == lax.broadcasted_iota(jnp.int32, (_N_BLK, 128), 1)
    ).astype(jnp.float32)

    for j in range(_NB):
        wait_w(j)
        if j >= _KO:
            out_copy(j - _KO).wait()
        chunk = b_ref[pl.ds(j * (_N_BLK // 128), _N_BLK // 128), :]
        spread = lax.dot_general(
            s_mat, chunk, (((1,), (0,)), ((), ())),
            preferred_element_type=jnp.float32,
        )
        bias_col = jnp.sum(spread * lane_mask, axis=1, keepdims=True)
        o_bufs[j % _KO] = (
            lax.dot_general(
                w_bufs[j % _KW], x, (((1,), (1,)), ((), ())),
                preferred_element_type=jnp.float32,
            )
            + bias_col
        )
        out_copy(j).start()
        if j + _KW < _NB:
            start_w(j + _KW)

    for j in range(max(_NB - _KO, 0), _NB):
        out_copy(j).wait()


def _tc_project(x, W, b_mat):
    return pl.pallas_call(
        _proj_body,
        in_specs=[
            pl.BlockSpec(memory_space=pltpu.VMEM),
            pl.BlockSpec(memory_space=pltpu.VMEM),
            pl.BlockSpec(memory_space=pltpu.HBM),
        ],
        out_specs=pl.BlockSpec(memory_space=pltpu.HBM),
        out_shape=jax.ShapeDtypeStruct((VOCAB, BATCH), jnp.float32),
        scratch_shapes=[
            pltpu.VMEM((_KW, _N_BLK, EMBED_DIM), jnp.float32),
            pltpu.VMEM((_KO, _N_BLK, BATCH), jnp.float32),
            pltpu.SemaphoreType.DMA((_KW,)),
            pltpu.SemaphoreType.DMA((_KO,)),
        ],
    )(x, b_mat, W)


@jax.jit
def kernel(prev_relation_id, relation_embeddings, W, b):
    idx = prev_relation_id.astype(jnp.int32)
    rows = _sc_gather(relation_embeddings, idx)
    b_mat = jnp.pad(b, (0, _BPAD - VOCAB)).reshape(_BPAD // 128, 128)
    scores_t = _tc_project(rows, W, b_mat)
    return scores_t.T


# speedup vs baseline: 1.1277x; 1.1181x over previous
import functools

import jax
import jax.numpy as jnp
from jax import lax
from jax.experimental import pallas as pl
from jax.experimental.pallas import tpu as pltpu
from jax.experimental.pallas import tpu_sc as plsc

VOCAB = 100000
EMBED_DIM = 64
BATCH = 1024


_NC, _NS = 2, 16
_NW = _NC * _NS
_B_PER_W = BATCH // _NW


def _sc_gather_body(table_hbm, idx_hbm, out_hbm, idx_s, stage, rows_v, sem):
    wid = lax.axis_index("s") * _NC + lax.axis_index("c")
    base = wid * _B_PER_W
    pltpu.sync_copy(idx_hbm.at[pl.ds(base, _B_PER_W)], idx_s)
    idx_sc = []
    for v in range(_B_PER_W // 16):
        vec = idx_s[pl.ds(16 * v, 16)]
        idx_sc.extend(vec[i] for i in range(16))
    copies = []
    for i in range(_B_PER_W):
        tile_base = pl.multiple_of((idx_sc[i] // 8) * 8, 8)
        cp = pltpu.make_async_copy(
            table_hbm.at[pl.ds(tile_base, 8)], stage.at[i], sem)
        cp.start()
        copies.append(cp)
    for cp in copies:
        cp.wait()
    for i in range(_B_PER_W):
        sub = idx_sc[i] % 8
        for k in range(EMBED_DIM // 16):
            rows_v[i, pl.ds(16 * k, 16)] = stage[i, sub, pl.ds(16 * k, 16)]
    pltpu.sync_copy(rows_v, out_hbm.at[pl.ds(base, _B_PER_W)])


def _sc_gather(table, idx):
    mesh = plsc.VectorSubcoreMesh(core_axis_name="c", subcore_axis_name="s")
    return pl.kernel(
        _sc_gather_body,
        out_type=jax.ShapeDtypeStruct((BATCH, EMBED_DIM), jnp.float32),
        mesh=mesh,
        scratch_types=[
            pltpu.VMEM((_B_PER_W,), jnp.int32),
            pltpu.VMEM((_B_PER_W, 8, EMBED_DIM), jnp.float32),
            pltpu.VMEM((_B_PER_W, EMBED_DIM), jnp.float32),
            pltpu.SemaphoreType.DMA,
        ],
    )(table, idx)


_N_BLK = 4096
_NB = pl.cdiv(VOCAB, _N_BLK)
_LAST = VOCAB - (_NB - 1) * _N_BLK
_KW = 2
_KO = 2
_BPAD = _NB * _N_BLK


def _proj_body(x_ref, b_ref, w_hbm, o_hbm, w_bufs, o_bufs, w_sem, o_sem):
    def rows(j):
        return _N_BLK if j < _NB - 1 else _LAST

    def start_w(j):
        pltpu.make_async_copy(
            w_hbm.at[pl.ds(j * _N_BLK, rows(j))],
            w_bufs.at[j % _KW, pl.ds(0, rows(j))],
            w_sem.at[j % _KW],
        ).start()

    def wait_w(j):
        pltpu.make_async_copy(
            w_hbm.at[pl.ds(j * _N_BLK, rows(j))],
            w_bufs.at[j % _KW, pl.ds(0, rows(j))],
            w_sem.at[j % _KW],
        ).wait()

    def out_copy(j):
        return pltpu.make_async_copy(
            o_bufs.at[j % _KO, pl.ds(0, rows(j))],
            o_hbm.at[pl.ds(j * _N_BLK, rows(j))],
            o_sem.at[j % _KO],
        )

    for j in range(min(_KW, _NB)):
        start_w(j)

    x = x_ref[...]
    nck = _N_BLK // 128
    vrow = lax.broadcasted_iota(jnp.int32, (_N_BLK, nck), 0)
    sel = (vrow // 128 == lax.broadcasted_iota(jnp.int32, (_N_BLK, nck), 1))
    s_mat = sel.astype(jnp.float32)
    vlane = lax.broadcasted_iota(jnp.int32, (_N_BLK, 128), 0) % 128
    lane_mask = (
        vlane == lax.broadcasted_iota(jnp.int32, (_N_BLK, 128), 1)
    ).astype(jnp.float32)

    for j in range(_NB):
        wait_w(j)
        if j >= _KO:
            out_copy(j - _KO).wait()
        chunk = b_ref[pl.ds(j * (_N_BLK // 128), _N_BLK // 128), :]
        spread = lax.dot_general(
            s_mat, chunk, (((1,), (0,)), ((), ())),
            preferred_element_type=jnp.float32,
        )
        bias_col = jnp.sum(spread * lane_mask, axis=1, keepdims=True)
        o_bufs[j % _KO] = (
            lax.dot_general(
                w_bufs[j % _KW], x, (((1,), (1,)), ((), ())),
                preferred_element_type=jnp.float32,
            )
            + bias_col
        )
        out_copy(j).start()
        if j + _KW < _NB:
            start_w(j + _KW)

    for j in range(max(_NB - _KO, 0), _NB):
        out_copy(j).wait()


def _tc_project(x, W, b_mat):
    return pl.pallas_call(
        _proj_body,
        in_specs=[
            pl.BlockSpec(memory_space=pltpu.VMEM),
            pl.BlockSpec(memory_space=pltpu.VMEM),
            pl.BlockSpec(memory_space=pltpu.HBM),
        ],
        out_specs=pl.BlockSpec(memory_space=pltpu.HBM),
        out_shape=jax.ShapeDtypeStruct((VOCAB, BATCH), jnp.float32),
        scratch_shapes=[
            pltpu.VMEM((_KW, _N_BLK, EMBED_DIM), jnp.float32),
            pltpu.VMEM((_KO, _N_BLK, BATCH), jnp.float32),
            pltpu.SemaphoreType.DMA((_KW,)),
            pltpu.SemaphoreType.DMA((_KO,)),
        ],
    )(x, b_mat, W)


@jax.jit
def kernel(prev_relation_id, relation_embeddings, W, b):
    idx = prev_relation_id.astype(jnp.int32)
    rows = _sc_gather(relation_embeddings, idx)
    b_mat = jnp.pad(b, (0, _BPAD - VOCAB)).reshape(_BPAD // 128, 128)
    scores_t = _tc_project(rows, W, b_mat)
    return scores_t.T
